# baseline (speedup 1.0000x reference)
import functools
import math

import jax
import jax.numpy as jnp
from jax.experimental import pallas as pl
from jax.experimental.pallas import tpu as pltpu


def _dot(a, b):
    return jax.lax.dot_general(a, b, (((1,), (0,)), ((), ())),
                               preferred_element_type=jnp.float32)


def _moe_body(cap, x_ref, wg_ref, ws_ref, be_ref, u_ref, out_ref, cnt_ref):
    i = pl.program_id(0)

    @pl.when(i == 0)
    def _():
        cnt_ref[...] = jnp.zeros_like(cnt_ref)

    xb_bf = x_ref[...]
    E = be_ref.shape[1]
    B = xb_bf.shape[1]

    logits = _dot(wg_ref[...], xb_bf)
    m = jnp.max(logits, axis=0, keepdims=True)
    g = jnp.exp(logits - m)
    gates = g / jnp.sum(g, axis=0, keepdims=True)
    gate = jnp.max(gates, axis=0, keepdims=True)

    s_iota = jax.lax.broadcasted_iota(jnp.int32, (E, B), 0)
    first = jnp.min(jnp.where(gates == gate, s_iota, E), axis=0, keepdims=True)
    mask = (s_iota == first).astype(jnp.float32)

    CH = 256
    mask_bf = mask.astype(jnp.bfloat16)
    u = u_ref[...]
    off = cnt_ref[...]
    pos_chunks = []
    for k in range(B // CH):
        pc = _dot(mask_bf[:, k * CH:(k + 1) * CH], u)
        pos_chunks.append(pc + (off - 1.0))
        off = off + pc[:, CH - 1:CH]
    cnt_ref[...] = off
    pos = jnp.concatenate(pos_chunks, axis=1)

    keep = jnp.sum(jnp.where(pos < cap, mask, 0.0), axis=0, keepdims=True)
    coef = mask * (gate * keep)

    y_all = _dot(ws_ref[...], xb_bf)
    d_out = be_ref.shape[0]
    acc = jnp.zeros((d_out, B), jnp.float32)
    for e in range(E):
        ye = y_all[24 * e:24 * e + d_out, :]
        acc = acc + coef[e:e + 1, :] * (ye + be_ref[:, e:e + 1])
    out_ref[...] = acc


def kernel(inputs, Wg, We, be):
    d = inputs.shape[-1]
    E = Wg.shape[1]
    x = inputs.reshape(-1, d)
    T = x.shape[0]
    cap = float(math.ceil(T / E))
    B = 8192
    nblocks = T // B

    x_T = x.T.astype(jnp.bfloat16)
    WeT = We.transpose(0, 2, 1)
    Wstack = jnp.pad(WeT, ((0, 0), (0, 24 - d), (0, 0)))
    Wstack = jnp.pad(Wstack.reshape(24 * E, d),
                     ((0, 128 - 24 * E), (0, 0))).astype(jnp.bfloat16)
    WgT = Wg.T.astype(jnp.bfloat16)
    beT = be.T
    u = jnp.triu(jnp.ones((256, 256), jnp.bfloat16))

    out_T = pl.pallas_call(
        functools.partial(_moe_body, cap),
        grid=(nblocks,),
        in_specs=[
            pl.BlockSpec((d, B), lambda i: (0, i)),
            pl.BlockSpec((E, d), lambda i: (0, 0)),
            pl.BlockSpec((128, d), lambda i: (0, 0)),
            pl.BlockSpec((d, E), lambda i: (0, 0)),
            pl.BlockSpec((256, 256), lambda i: (0, 0)),
        ],
        out_specs=pl.BlockSpec((d, B), lambda i: (0, i)),
        out_shape=jax.ShapeDtypeStruct((d, T), jnp.float32),
        scratch_shapes=[pltpu.VMEM((E, 1), jnp.float32)],
        compiler_params=pltpu.CompilerParams(
            dimension_semantics=("arbitrary",)),
    )(x_T, WgT, Wstack, beT, u)
    return out_T.T.reshape(inputs.shape)


# speedup vs baseline: 1.0149x; 1.0149x over previous
import functools
import math

import jax
import jax.numpy as jnp
from jax.experimental import pallas as pl
from jax.experimental.pallas import tpu as pltpu


def _dot(a, b):
    return jax.lax.dot_general(a, b, (((1,), (0,)), ((), ())),
                               preferred_element_type=jnp.float32)


def _moe_body(cap, x_ref, wg_ref, ws_ref, be_ref, u_ref, out_ref, cnt_ref):
    i = pl.program_id(0)

    @pl.when(i == 0)
    def _():
        cnt_ref[...] = jnp.zeros_like(cnt_ref)

    xb_bf = x_ref[...]
    E = be_ref.shape[1]
    B = xb_bf.shape[1]

    logits = _dot(wg_ref[...], xb_bf)
    m = jnp.max(logits, axis=0, keepdims=True)
    g = jnp.exp(logits - m)
    s = jnp.sum(g, axis=0, keepdims=True)
    gates = g / s
    gate = 1.0 / s

    s_iota = jax.lax.broadcasted_iota(jnp.int32, (E, B), 0)
    first = jnp.min(jnp.where(gates == gate, s_iota, E), axis=0, keepdims=True)
    mask = (s_iota == first).astype(jnp.float32)

    CH = 256
    mask_bf = mask.astype(jnp.bfloat16)
    u = u_ref[...]
    off = cnt_ref[...]
    pos_chunks = []
    for k in range(B // CH):
        pc = _dot(mask_bf[:, k * CH:(k + 1) * CH], u)
        pos_chunks.append(pc + (off - 1.0))
        off = off + pc[:, CH - 1:CH]
    cnt_ref[...] = off
    pos = jnp.concatenate(pos_chunks, axis=1)

    coef = jnp.where(pos < cap, mask, 0.0) * gate

    y_all = _dot(ws_ref[...], xb_bf)
    d_out = be_ref.shape[0]
    acc = jnp.zeros((d_out, B), jnp.float32)
    for e in range(E):
        ye = y_all[24 * e:24 * e + d_out, :]
        acc = acc + coef[e:e + 1, :] * (ye + be_ref[:, e:e + 1])
    out_ref[...] = acc


def kernel(inputs, Wg, We, be):
    d = inputs.shape[-1]
    E = Wg.shape[1]
    x = inputs.reshape(-1, d)
    T = x.shape[0]
    cap = float(math.ceil(T / E))
    B = 16384
    nblocks = T // B

    x_T = x.T.astype(jnp.bfloat16)
    WeT = We.transpose(0, 2, 1)
    Wstack = jnp.pad(WeT, ((0, 0), (0, 24 - d), (0, 0)))
    Wstack = jnp.pad(Wstack.reshape(24 * E, d),
                     ((0, 128 - 24 * E), (0, 0))).astype(jnp.bfloat16)
    WgT = Wg.T.astype(jnp.bfloat16)
    beT = be.T
    u = jnp.triu(jnp.ones((256, 256), jnp.bfloat16))

    out_T = pl.pallas_call(
        functools.partial(_moe_body, cap),
        grid=(nblocks,),
        in_specs=[
            pl.BlockSpec((d, B), lambda i: (0, i)),
            pl.BlockSpec((E, d), lambda i: (0, 0)),
            pl.BlockSpec((128, d), lambda i: (0, 0)),
            pl.BlockSpec((d, E), lambda i: (0, 0)),
            pl.BlockSpec((256, 256), lambda i: (0, 0)),
        ],
        out_specs=pl.BlockSpec((d, B), lambda i: (0, i)),
        out_shape=jax.ShapeDtypeStruct((d, T), jnp.float32),
        scratch_shapes=[pltpu.VMEM((E, 1), jnp.float32)],
        compiler_params=pltpu.CompilerParams(
            dimension_semantics=("arbitrary",)),
    )(x_T, WgT, Wstack, beT, u)
    return out_T.T.reshape(inputs.shape)
